# baseline (speedup 1.0000x reference)
import functools

import jax
import jax.numpy as jnp
from jax import lax
from jax.experimental import pallas as pl
from jax.experimental.pallas import tpu as pltpu
from jax.experimental.pallas import tpu_sc as plsc

N = 10000
E = 320000
F = 128
H = 128
R = 4
L = 12

NC = 2
NS = 16
NW = NC * NS
CHUNK = 128
NCHUNK = 80
EPT = NCHUNK * CHUNK
REAL_EPT = E // NW
ROWS_PT = N // NS
ACC_ROWS = N + 8


def _sc_body(allh_hbm, gidx_hbm, dst_hbm, zeros_hbm, out_hbm,
             gcb, dcb, rows0, rows1, acc,
             gsem0, gsem1, igsem0, igsem1, idsem0, idsem1):
    c = lax.axis_index("c")
    s = lax.axis_index("s")
    wid = c * NS + s
    base = wid * EPT

    pltpu.sync_copy(zeros_hbm, acc.at[pl.ds(s * ROWS_PT, ROWS_PT)])
    plsc.subcore_barrier()

    igsem = (igsem0, igsem1)
    idsem = (idsem0, idsem1)

    def idx_start(j, p):
        off = pl.ds(base + j * CHUNK, CHUNK)
        pltpu.async_copy(gidx_hbm.at[off], gcb.at[p], igsem[p])
        pltpu.async_copy(dst_hbm.at[off], dcb.at[p], idsem[p])

    def idx_wait(p):
        off = pl.ds(base, CHUNK)
        pltpu.make_async_copy(gidx_hbm.at[off], gcb.at[p], igsem[p]).wait()
        pltpu.make_async_copy(dst_hbm.at[off], dcb.at[p], idsem[p]).wait()

    def g_start(p, rbuf, sem):
        pltpu.async_copy(allh_hbm.at[gcb.at[p]], rbuf, sem)

    def g_wait(p, rbuf, sem):
        pltpu.make_async_copy(allh_hbm.at[gcb.at[p]], rbuf, sem).wait()

    def scat(p, rbuf):
        pltpu.sync_copy(rbuf, acc.at[dcb.at[p]], add=True)

    idx_start(0, 0)
    idx_start(1, 1)
    idx_wait(0)
    g_start(0, rows0, gsem0)

    def pair(i, carry):
        j0 = 2 * i
        g_wait(0, rows0, gsem0)
        idx_wait(1)
        g_start(1, rows1, gsem1)
        scat(0, rows0)

        @pl.when(j0 + 2 < NCHUNK)
        def _():
            idx_start(j0 + 2, 0)

        g_wait(1, rows1, gsem1)

        @pl.when(j0 + 2 < NCHUNK)
        def _():
            idx_wait(0)
            g_start(0, rows0, gsem0)

        scat(1, rows1)

        @pl.when(j0 + 3 < NCHUNK)
        def _():
            idx_start(j0 + 3, 1)

        return carry

    lax.fori_loop(0, NCHUNK // 2, pair, 0)
    plsc.subcore_barrier()

    pltpu.sync_copy(acc.at[pl.ds(s * ROWS_PT, ROWS_PT)], out_hbm.at[c, s])


@functools.cache
def _sc_aggregate():
    return pl.kernel(
        _sc_body,
        out_type=jax.ShapeDtypeStruct((NC, NS, ROWS_PT, H), jnp.float32),
        mesh=plsc.VectorSubcoreMesh(core_axis_name="c", subcore_axis_name="s",
                                    num_cores=NC, num_subcores=NS),
        scratch_types=[
            pltpu.VMEM((2, CHUNK), jnp.int32),
            pltpu.VMEM((2, CHUNK), jnp.int32),
            pltpu.VMEM((CHUNK, H), jnp.float32),
            pltpu.VMEM((CHUNK, H), jnp.float32),
            pltpu.VMEM_SHARED((ACC_ROWS, H), jnp.float32),
            pltpu.SemaphoreType.DMA,
            pltpu.SemaphoreType.DMA,
            pltpu.SemaphoreType.DMA,
            pltpu.SemaphoreType.DMA,
            pltpu.SemaphoreType.DMA,
            pltpu.SemaphoreType.DMA,
        ],
    )


BN = 2000
_GRID = N // BN


def _dot(a, b):
    return jnp.dot(a, b, preferred_element_type=jnp.float32)


def _init_body(x_ref, win_ref, bin_ref, wrel_ref, h_ref, allh_ref):
    h = jnp.tanh(_dot(x_ref[...], win_ref[...]) + bin_ref[...])
    h_ref[...] = h
    for r in range(R):
        allh_ref[r] = _dot(h, wrel_ref[r])


def _update(h, p_ref, wself_ref, brel_ref, wu1a_ref, wu1b_ref, bu1_ref,
            wu2a_ref, wu2b_ref, bu2_ref):
    agg = p_ref[0] + p_ref[1]
    msg = jnp.tanh(agg + _dot(h, wself_ref[...]) + brel_ref[...])
    mid = jnp.tanh(_dot(h, wu1a_ref[...]) + _dot(msg, wu1b_ref[...])
                   + bu1_ref[...])
    return jnp.tanh(_dot(h, wu2a_ref[...]) + _dot(mid, wu2b_ref[...])
                    + bu2_ref[...])


def _mid_body(h_ref, p_ref, wself_ref, brel_ref, wu1a_ref, wu1b_ref, bu1_ref,
              wu2a_ref, wu2b_ref, bu2_ref, wrel_ref, hn_ref, allh_ref):
    hn = _update(h_ref[...], p_ref, wself_ref, brel_ref, wu1a_ref, wu1b_ref,
                 bu1_ref, wu2a_ref, wu2b_ref, bu2_ref)
    hn_ref[...] = hn
    for r in range(R):
        allh_ref[r] = _dot(hn, wrel_ref[r])


def _last_body(h_ref, p_ref, wself_ref, brel_ref, wu1a_ref, wu1b_ref, bu1_ref,
               wu2a_ref, wu2b_ref, bu2_ref, hn_ref):
    hn_ref[...] = _update(h_ref[...], p_ref, wself_ref, brel_ref, wu1a_ref,
                          wu1b_ref, bu1_ref, wu2a_ref, wu2b_ref, bu2_ref)


def _row_spec(width=H):
    return pl.BlockSpec((BN, width), lambda i: (i, 0))


def _full_spec(shape):
    nd = len(shape)
    return pl.BlockSpec(shape, lambda i, _n=nd: (0,) * _n)


_P_SPEC = pl.BlockSpec((NC, BN, H), lambda i: (0, i, 0))
_ALLH_SPEC = pl.BlockSpec((R, BN, H), lambda i: (0, i, 0))

_W_SPECS = [
    _full_spec((H, H)),
    _full_spec((1, H)),
    _full_spec((H, 2 * H)),
    _full_spec((H, 2 * H)),
    _full_spec((1, 2 * H)),
    _full_spec((H, H)),
    _full_spec((2 * H, H)),
    _full_spec((1, H)),
]

_H_OUT = jax.ShapeDtypeStruct((N, H), jnp.float32)
_ALLH_OUT = jax.ShapeDtypeStruct((R, N, H), jnp.float32)

_init_call = pl.pallas_call(
    _init_body,
    grid=(_GRID,),
    in_specs=[_row_spec(F), _full_spec((F, H)), _full_spec((1, H)),
              _full_spec((R, H, H))],
    out_specs=[_row_spec(), _ALLH_SPEC],
    out_shape=[_H_OUT, _ALLH_OUT],
)

_mid_call = pl.pallas_call(
    _mid_body,
    grid=(_GRID,),
    in_specs=[_row_spec(), _P_SPEC] + _W_SPECS + [_full_spec((R, H, H))],
    out_specs=[_row_spec(), _ALLH_SPEC],
    out_shape=[_H_OUT, _ALLH_OUT],
)

_last_call = pl.pallas_call(
    _last_body,
    grid=(_GRID,),
    in_specs=[_row_spec(), _P_SPEC] + _W_SPECS,
    out_specs=_row_spec(),
    out_shape=_H_OUT,
)


def kernel(x, edge_index, edge_type, W_in, b_in, W_rel, W_self, b_rel,
           W_up1, b_up1, W_up2, b_up2):
    src = edge_index[0]
    dst = edge_index[1]
    gidx = jnp.concatenate(
        [(edge_type * N + src).reshape(NW, REAL_EPT),
         jnp.zeros((NW, EPT - REAL_EPT), jnp.int32)], axis=1).reshape(-1)
    dst2 = jnp.concatenate(
        [dst.reshape(NW, REAL_EPT),
         jnp.full((NW, EPT - REAL_EPT), N, jnp.int32)], axis=1).reshape(-1)
    zeros = jnp.zeros((ROWS_PT, H), jnp.float32)

    b_in2 = b_in.reshape(1, H)
    brel2 = b_rel.reshape(L, 1, H)
    bu12 = b_up1.reshape(L, 1, 2 * H)
    bu22 = b_up2.reshape(L, 1, H)
    wu1a = W_up1[:, :H, :]
    wu1b = W_up1[:, H:, :]
    wu2a = W_up2[:, :H, :]
    wu2b = W_up2[:, H:, :]

    h, all_h = _init_call(x, W_in, b_in2, W_rel[0])
    for l in range(L):
        partials = _sc_aggregate()(all_h.reshape(R * N, H), gidx, dst2,
                                   zeros).reshape(NC, N, H)
        wargs = (W_self[l], brel2[l], wu1a[l], wu1b[l], bu12[l],
                 wu2a[l], wu2b[l], bu22[l])
        if l < L - 1:
            h, all_h = _mid_call(h, partials, *wargs, W_rel[l + 1])
        else:
            h = _last_call(h, partials, *wargs)
    return h


# speedup vs baseline: 3.0692x; 3.0692x over previous
import functools

import jax
import jax.numpy as jnp
from jax import lax
from jax.experimental import pallas as pl
from jax.experimental.pallas import tpu as pltpu
from jax.experimental.pallas import tpu_sc as plsc

N = 10000
E = 320000
F = 128
H = 128
R = 4
L = 12

NC = 2
NS = 16
NW = NC * NS
CHUNK = 125
NCHUNK = 80
HALF = NCHUNK // 2
EPT = NCHUNK * CHUNK
ROWS_PT = N // NS


def _sc_body(allh_hbm, gidx_hbm, dst_hbm, zeros_hbm, out_hbm,
             gbuf, dbuf, rows0, rows1, acc, gsem0, gsem1):
    c = lax.axis_index("c")
    s = lax.axis_index("s")
    wid = c * NS + s

    pltpu.sync_copy(zeros_hbm, acc.at[pl.ds(s * ROWS_PT, ROWS_PT)])
    plsc.subcore_barrier()

    def g_start(j, rbuf, sem):
        pltpu.async_copy(allh_hbm.at[gbuf.at[j]], rbuf, sem)

    def g_wait(rbuf, sem):
        pltpu.make_async_copy(allh_hbm.at[gbuf.at[0]], rbuf, sem).wait()

    def scat(j, rbuf):
        pltpu.sync_copy(rbuf, acc.at[dbuf.at[j]], add=True)

    for phase in range(2):
        half = pl.ds(phase * HALF, HALF)
        pltpu.sync_copy(gidx_hbm.at[wid, half], gbuf)
        pltpu.sync_copy(dst_hbm.at[wid, half], dbuf)
        g_start(0, rows0, gsem0)

        def pair(i, carry):
            j0 = 2 * i
            g_wait(rows0, gsem0)
            g_start(j0 + 1, rows1, gsem1)
            scat(j0, rows0)

            @pl.when(j0 + 2 < HALF)
            def _():
                g_start(j0 + 2, rows0, gsem0)

            g_wait(rows1, gsem1)
            scat(j0 + 1, rows1)
            return carry

        lax.fori_loop(0, HALF // 2, pair, 0)
    plsc.subcore_barrier()

    pltpu.sync_copy(acc.at[pl.ds(s * ROWS_PT, ROWS_PT)], out_hbm.at[c, s])


@functools.cache
def _sc_aggregate():
    return pl.kernel(
        _sc_body,
        out_type=jax.ShapeDtypeStruct((NC, NS, ROWS_PT, H), jnp.float32),
        mesh=plsc.VectorSubcoreMesh(core_axis_name="c", subcore_axis_name="s",
                                    num_cores=NC, num_subcores=NS),
        scratch_types=[
            pltpu.VMEM((HALF, CHUNK), jnp.int32),
            pltpu.VMEM((HALF, CHUNK), jnp.int32),
            pltpu.VMEM((CHUNK, H), jnp.float32),
            pltpu.VMEM((CHUNK, H), jnp.float32),
            pltpu.VMEM_SHARED((N, H), jnp.float32),
            pltpu.SemaphoreType.DMA,
            pltpu.SemaphoreType.DMA,
        ],
    )


BN = 2000
_GRID = N // BN


def _dot(a, b):
    return jnp.dot(a, b, preferred_element_type=jnp.float32)


def _init_body(x_ref, win_ref, bin_ref, wrel_ref, h_ref, allh_ref):
    h = jnp.tanh(_dot(x_ref[...], win_ref[...]) + bin_ref[...])
    h_ref[...] = h
    for r in range(R):
        allh_ref[r] = _dot(h, wrel_ref[r])


def _update(h, p_ref, wself_ref, brel_ref, wu1a_ref, wu1b_ref, bu1_ref,
            wu2a_ref, wu2b_ref, bu2_ref):
    agg = p_ref[0] + p_ref[1]
    msg = jnp.tanh(agg + _dot(h, wself_ref[...]) + brel_ref[...])
    mid = jnp.tanh(_dot(h, wu1a_ref[...]) + _dot(msg, wu1b_ref[...])
                   + bu1_ref[...])
    return jnp.tanh(_dot(h, wu2a_ref[...]) + _dot(mid, wu2b_ref[...])
                    + bu2_ref[...])


def _mid_body(h_ref, p_ref, wself_ref, brel_ref, wu1a_ref, wu1b_ref, bu1_ref,
              wu2a_ref, wu2b_ref, bu2_ref, wrel_ref, hn_ref, allh_ref):
    hn = _update(h_ref[...], p_ref, wself_ref, brel_ref, wu1a_ref, wu1b_ref,
                 bu1_ref, wu2a_ref, wu2b_ref, bu2_ref)
    hn_ref[...] = hn
    for r in range(R):
        allh_ref[r] = _dot(hn, wrel_ref[r])


def _last_body(h_ref, p_ref, wself_ref, brel_ref, wu1a_ref, wu1b_ref, bu1_ref,
               wu2a_ref, wu2b_ref, bu2_ref, hn_ref):
    hn_ref[...] = _update(h_ref[...], p_ref, wself_ref, brel_ref, wu1a_ref,
                          wu1b_ref, bu1_ref, wu2a_ref, wu2b_ref, bu2_ref)


def _row_spec(width=H):
    return pl.BlockSpec((BN, width), lambda i: (i, 0))


def _full_spec(shape):
    nd = len(shape)
    return pl.BlockSpec(shape, lambda i, _n=nd: (0,) * _n)


_P_SPEC = pl.BlockSpec((NC, BN, H), lambda i: (0, i, 0))
_ALLH_SPEC = pl.BlockSpec((R, BN, H), lambda i: (0, i, 0))

_W_SPECS = [
    _full_spec((H, H)),
    _full_spec((1, H)),
    _full_spec((H, 2 * H)),
    _full_spec((H, 2 * H)),
    _full_spec((1, 2 * H)),
    _full_spec((H, H)),
    _full_spec((2 * H, H)),
    _full_spec((1, H)),
]

_H_OUT = jax.ShapeDtypeStruct((N, H), jnp.float32)
_ALLH_OUT = jax.ShapeDtypeStruct((R, N, H), jnp.float32)

_init_call = pl.pallas_call(
    _init_body,
    grid=(_GRID,),
    in_specs=[_row_spec(F), _full_spec((F, H)), _full_spec((1, H)),
              _full_spec((R, H, H))],
    out_specs=[_row_spec(), _ALLH_SPEC],
    out_shape=[_H_OUT, _ALLH_OUT],
)

_mid_call = pl.pallas_call(
    _mid_body,
    grid=(_GRID,),
    in_specs=[_row_spec(), _P_SPEC] + _W_SPECS + [_full_spec((R, H, H))],
    out_specs=[_row_spec(), _ALLH_SPEC],
    out_shape=[_H_OUT, _ALLH_OUT],
)

_last_call = pl.pallas_call(
    _last_body,
    grid=(_GRID,),
    in_specs=[_row_spec(), _P_SPEC] + _W_SPECS,
    out_specs=_row_spec(),
    out_shape=_H_OUT,
)


def kernel(x, edge_index, edge_type, W_in, b_in, W_rel, W_self, b_rel,
           W_up1, b_up1, W_up2, b_up2):
    src = edge_index[0]
    dst = edge_index[1]
    gidx = (edge_type * N + src).reshape(NW, NCHUNK, CHUNK)
    dst2 = dst.reshape(NW, NCHUNK, CHUNK)
    zeros = jnp.zeros((ROWS_PT, H), jnp.float32)

    b_in2 = b_in.reshape(1, H)
    brel2 = b_rel.reshape(L, 1, H)
    bu12 = b_up1.reshape(L, 1, 2 * H)
    bu22 = b_up2.reshape(L, 1, H)
    wu1a = W_up1[:, :H, :]
    wu1b = W_up1[:, H:, :]
    wu2a = W_up2[:, :H, :]
    wu2b = W_up2[:, H:, :]

    h, all_h = _init_call(x, W_in, b_in2, W_rel[0])
    for l in range(L):
        partials = _sc_aggregate()(all_h.reshape(R * N, H), gidx, dst2,
                                   zeros).reshape(NC, N, H)
        wargs = (W_self[l], brel2[l], wu1a[l], wu1b[l], bu12[l],
                 wu2a[l], wu2b[l], bu22[l])
        if l < L - 1:
            h, all_h = _mid_call(h, partials, *wargs, W_rel[l + 1])
        else:
            h = _last_call(h, partials, *wargs)
    return h
